# baseline (speedup 1.0000x reference)
import jax
import jax.numpy as jnp
from jax.experimental import pallas as pl
from jax.experimental.pallas import tpu as pltpu

_B = 16384
_H = 1024
_X = 2624
_BLOCK = 512


def _gate_half(c, h, x, wc_ref, wh_ref, wx_ref, b_ref, o0_ref, o1_ref, o2_ref):
    logits = jnp.dot(c, wc_ref[...], preferred_element_type=jnp.float32)
    logits += jnp.dot(h, wh_ref[...], preferred_element_type=jnp.float32)
    logits += jnp.dot(x, wx_ref[...], preferred_element_type=jnp.float32)
    logits += b_ref[...]
    m = jnp.max(logits, axis=1, keepdims=True)
    e = jnp.exp(logits - m)
    s = jnp.sum(e, axis=1, keepdims=True)
    w = e / s
    o0_ref[...] = w[:, 0]
    o1_ref[...] = w[:, 1]
    o2_ref[...] = w[:, 2]


def _gate_body(c0_ref, h0_ref, x0_ref, c1_ref, h1_ref, x1_ref,
               wc_ref, wh_ref, wx_ref, b_ref,
               o00_ref, o01_ref, o02_ref, o10_ref, o11_ref, o12_ref):
    _gate_half(c0_ref[...], h0_ref[...], x0_ref[...], wc_ref, wh_ref, wx_ref,
               b_ref, o00_ref, o01_ref, o02_ref)
    _gate_half(c1_ref[...], h1_ref[...], x1_ref[...], wc_ref, wh_ref, wx_ref,
               b_ref, o10_ref, o11_ref, o12_ref)


def kernel(c_img, h_t, x_t, W, b):
    wc = W[:, :_H].T
    wh = W[:, _H:2 * _H].T
    wx = W[:, 2 * _H:].T
    b2 = b.reshape(1, 3)
    n = _B // _BLOCK
    grid = (n // 2,)
    outs = pl.pallas_call(
        _gate_body,
        grid=grid,
        in_specs=[
            pl.BlockSpec((_BLOCK, _H), lambda i: (2 * i, 0)),
            pl.BlockSpec((_BLOCK, _H), lambda i: (2 * i, 0)),
            pl.BlockSpec((_BLOCK, _X), lambda i: (2 * i, 0)),
            pl.BlockSpec((_BLOCK, _H), lambda i: (2 * i + 1, 0)),
            pl.BlockSpec((_BLOCK, _H), lambda i: (2 * i + 1, 0)),
            pl.BlockSpec((_BLOCK, _X), lambda i: (2 * i + 1, 0)),
            pl.BlockSpec((_H, 3), lambda i: (0, 0)),
            pl.BlockSpec((_H, 3), lambda i: (0, 0)),
            pl.BlockSpec((_X, 3), lambda i: (0, 0)),
            pl.BlockSpec((1, 3), lambda i: (0, 0)),
        ],
        out_specs=[pl.BlockSpec((_BLOCK,), lambda i: (i,))] * 6,
        out_shape=[jax.ShapeDtypeStruct((_B // 2,), jnp.float32)] * 6,
    )(c_img, h_t, x_t, c_img, h_t, x_t, wc, wh, wx, b2)

    def _interleave(a, c):
        m = _B // (2 * _BLOCK)
        return jnp.stack([a.reshape(m, _BLOCK), c.reshape(m, _BLOCK)],
                         axis=1).reshape(_B)

    return (_interleave(outs[0], outs[3]),
            _interleave(outs[1], outs[4]),
            _interleave(outs[2], outs[5]))


# speedup vs baseline: 1.0075x; 1.0075x over previous
import functools

import jax
import jax.numpy as jnp
from jax.experimental import pallas as pl
from jax.experimental.pallas import tpu as pltpu

_B = 16384
_H = 1024
_X = 2624
_CHUNK = 512
_NCHUNK = _B // _CHUNK
_NBUF = 4


def _gate_body(c_hbm, h_hbm, x_hbm, wc_ref, wh_ref, wx_ref, b_ref,
               o0_ref, o1_ref, o2_ref,
               cbuf, hbuf, xbuf, csem, hsem, xsem):
    def start(slot, chunk):
        pltpu.make_async_copy(
            c_hbm.at[pl.ds(chunk * _CHUNK, _CHUNK), :], cbuf.at[slot],
            csem.at[slot]).start()
        pltpu.make_async_copy(
            h_hbm.at[pl.ds(chunk * _CHUNK, _CHUNK), :], hbuf.at[slot],
            hsem.at[slot]).start()
        pltpu.make_async_copy(
            x_hbm.at[pl.ds(chunk * _CHUNK, _CHUNK), :], xbuf.at[slot],
            xsem.at[slot]).start()

    def wait(slot, chunk):
        pltpu.make_async_copy(
            c_hbm.at[pl.ds(chunk * _CHUNK, _CHUNK), :], cbuf.at[slot],
            csem.at[slot]).wait()
        pltpu.make_async_copy(
            h_hbm.at[pl.ds(chunk * _CHUNK, _CHUNK), :], hbuf.at[slot],
            hsem.at[slot]).wait()
        pltpu.make_async_copy(
            x_hbm.at[pl.ds(chunk * _CHUNK, _CHUNK), :], xbuf.at[slot],
            xsem.at[slot]).wait()

    for i in range(_NBUF - 1):
        start(i, i)

    def step(i, _):
        slot = jax.lax.rem(i, _NBUF)
        nxt = i + (_NBUF - 1)

        @pl.when(nxt < _NCHUNK)
        def _():
            start(jax.lax.rem(nxt, _NBUF), nxt)

        wait(slot, i)
        logits = jnp.dot(cbuf[slot], wc_ref[...],
                         preferred_element_type=jnp.float32)
        logits += jnp.dot(hbuf[slot], wh_ref[...],
                          preferred_element_type=jnp.float32)
        logits += jnp.dot(xbuf[slot], wx_ref[...],
                          preferred_element_type=jnp.float32)
        logits += b_ref[...]
        m = jnp.max(logits, axis=1, keepdims=True)
        e = jnp.exp(logits - m)
        s = jnp.sum(e, axis=1, keepdims=True)
        w = e / s
        base = i * _CHUNK
        o0_ref[pl.ds(base, _CHUNK)] = w[:, 0]
        o1_ref[pl.ds(base, _CHUNK)] = w[:, 1]
        o2_ref[pl.ds(base, _CHUNK)] = w[:, 2]
        return ()

    jax.lax.fori_loop(0, _NCHUNK, step, ())


def kernel(c_img, h_t, x_t, W, b):
    wc = W[:, :_H].T
    wh = W[:, _H:2 * _H].T
    wx = W[:, 2 * _H:].T
    b2 = b.reshape(1, 3)
    outs = pl.pallas_call(
        _gate_body,
        in_specs=[
            pl.BlockSpec(memory_space=pl.ANY),
            pl.BlockSpec(memory_space=pl.ANY),
            pl.BlockSpec(memory_space=pl.ANY),
            pl.BlockSpec((_H, 3), lambda: (0, 0)),
            pl.BlockSpec((_H, 3), lambda: (0, 0)),
            pl.BlockSpec((_X, 3), lambda: (0, 0)),
            pl.BlockSpec((1, 3), lambda: (0, 0)),
        ],
        out_specs=[pl.BlockSpec((_B,), lambda: (0,))] * 3,
        out_shape=[jax.ShapeDtypeStruct((_B,), jnp.float32)] * 3,
        scratch_shapes=[
            pltpu.VMEM((_NBUF, _CHUNK, _H), jnp.float32),
            pltpu.VMEM((_NBUF, _CHUNK, _H), jnp.float32),
            pltpu.VMEM((_NBUF, _CHUNK, _X), jnp.float32),
            pltpu.SemaphoreType.DMA((_NBUF,)),
            pltpu.SemaphoreType.DMA((_NBUF,)),
            pltpu.SemaphoreType.DMA((_NBUF,)),
        ],
    )(c_img, h_t, x_t, wc, wh, wx, b2)
    return tuple(outs)


# speedup vs baseline: 2.6002x; 2.5807x over previous
import jax
import jax.numpy as jnp
from jax.experimental import pallas as pl

_B = 16384
_H = 1024
_X = 2624
_BLOCK = 1024


def _gate_body(c_ref, h_ref, xt_ref, wc_ref, wh_ref, wx_ref, b_ref,
               o0_ref, o1_ref, o2_ref):
    logits = jnp.dot(c_ref[...], wc_ref[...], preferred_element_type=jnp.float32)
    logits += jnp.dot(h_ref[...], wh_ref[...], preferred_element_type=jnp.float32)
    lx = jnp.dot(wx_ref[...], xt_ref[...], preferred_element_type=jnp.float32)
    logits += lx.T
    logits += b_ref[...]
    m = jnp.max(logits, axis=1, keepdims=True)
    e = jnp.exp(logits - m)
    s = jnp.sum(e, axis=1, keepdims=True)
    w = e / s
    o0_ref[...] = w[:, 0]
    o1_ref[...] = w[:, 1]
    o2_ref[...] = w[:, 2]


def kernel(c_img, h_t, x_t, W, b):
    wc = W[:, :_H].T
    wh = W[:, _H:2 * _H].T
    wx = W[:, 2 * _H:]
    b2 = b.reshape(1, 3)
    xt = x_t.T
    grid = (_B // _BLOCK,)
    outs = pl.pallas_call(
        _gate_body,
        grid=grid,
        in_specs=[
            pl.BlockSpec((_BLOCK, _H), lambda i: (i, 0)),
            pl.BlockSpec((_BLOCK, _H), lambda i: (i, 0)),
            pl.BlockSpec((_X, _BLOCK), lambda i: (0, i)),
            pl.BlockSpec((_H, 3), lambda i: (0, 0)),
            pl.BlockSpec((_H, 3), lambda i: (0, 0)),
            pl.BlockSpec((3, _X), lambda i: (0, 0)),
            pl.BlockSpec((1, 3), lambda i: (0, 0)),
        ],
        out_specs=[pl.BlockSpec((_BLOCK,), lambda i: (i,))] * 3,
        out_shape=[jax.ShapeDtypeStruct((_B,), jnp.float32)] * 3,
    )(c_img, h_t, xt, wc, wh, wx, b2)
    return tuple(outs)
